# baseline (speedup 1.0000x reference)
import functools

import jax
import jax.numpy as jnp
from jax import lax
from jax.experimental import pallas as pl
from jax.experimental.pallas import tpu as pltpu
from jax.experimental.pallas import tpu_sc as plsc

BATCH = 1024
HIDDEN = 128
TILE_V = 4096
NBUF = 3


def _sc_gather(input_ids, emb_table):
    info = plsc.get_sparse_core_info()
    nc, ns = info.num_cores, info.num_subcores
    nw = nc * ns
    b_per_w = BATCH // nw
    mesh = plsc.VectorSubcoreMesh(core_axis_name="c", subcore_axis_name="s")

    @functools.partial(
        pl.kernel,
        mesh=mesh,
        out_type=jax.ShapeDtypeStruct((BATCH, HIDDEN), jnp.float32),
        scratch_types=[
            pltpu.VMEM((b_per_w,), jnp.int32),
            pltpu.VMEM((b_per_w, HIDDEN), jnp.float32),
            pltpu.SemaphoreType.DMA,
            pltpu.SemaphoreType.DMA,
            pltpu.SemaphoreType.DMA,
            pltpu.SemaphoreType.DMA,
        ],
    )
    def gather_k(idx_hbm, table_hbm, out_hbm, idx_v, rows_v,
                 sem0, sem1, sem2, sem3):
        wid = lax.axis_index("s") * nc + lax.axis_index("c")
        base = wid * b_per_w
        h = b_per_w // 2
        pltpu.sync_copy(idx_hbm.at[pl.ds(base, b_per_w)], idx_v)
        c0 = pltpu.async_copy(
            table_hbm.at[idx_v.at[pl.ds(0, h)]], rows_v.at[pl.ds(0, h)], sem0)
        c1 = pltpu.async_copy(
            table_hbm.at[idx_v.at[pl.ds(h, h)]], rows_v.at[pl.ds(h, h)], sem1)
        c0.wait()
        w0 = pltpu.async_copy(
            rows_v.at[pl.ds(0, h)], out_hbm.at[pl.ds(base, h)], sem2)
        c1.wait()
        w1 = pltpu.async_copy(
            rows_v.at[pl.ds(h, h)], out_hbm.at[pl.ds(base + h, h)], sem3)
        w0.wait()
        w1.wait()

    return gather_k(input_ids, emb_table)


def _rows_of(grid, tail, j):
    return tail if (tail and j == grid - 1) else TILE_V


def _mm_body(grid, tail, x_ref, w_ref, b_ref, out_hbm, obuf, sems):
    i = pl.program_id(0)
    slot = lax.rem(i, NBUF)

    @pl.when(i >= NBUF)
    def _():
        pltpu.make_async_copy(
            obuf.at[slot], out_hbm.at[pl.ds(0, TILE_V)], sems.at[slot]
        ).wait()

    obuf.at[slot][...] = lax.dot_general(
        w_ref[...], x_ref[...],
        dimension_numbers=(((1,), (1,)), ((), ())),
        preferred_element_type=jnp.float32,
    ) + b_ref[...][:, None]

    @pl.when(i < grid - 1)
    def _():
        pltpu.make_async_copy(
            obuf.at[slot],
            out_hbm.at[pl.ds(i * TILE_V, TILE_V)],
            sems.at[slot],
        ).start()

    @pl.when(i == grid - 1)
    def _():
        rows = _rows_of(grid, tail, grid - 1)
        pltpu.make_async_copy(
            obuf.at[slot].at[pl.ds(0, rows)],
            out_hbm.at[pl.ds(i * TILE_V, rows)],
            sems.at[slot],
        ).start()
        for j in range(max(grid - NBUF, 0), grid):
            r = _rows_of(grid, tail, j)
            pltpu.make_async_copy(
                obuf.at[j % NBUF].at[pl.ds(0, r)],
                out_hbm.at[pl.ds(0, r)],
                sems.at[j % NBUF],
            ).wait()


def kernel(input_ids, emb_table, W, b):
    ids = input_ids.astype(jnp.int32)
    x = _sc_gather(ids, emb_table)

    vocab = W.shape[0]
    grid = (vocab + TILE_V - 1) // TILE_V
    tail = vocab % TILE_V

    out_t = pl.pallas_call(
        functools.partial(_mm_body, grid, tail),
        grid=(grid,),
        in_specs=[
            pl.BlockSpec((BATCH, HIDDEN), lambda i: (0, 0)),
            pl.BlockSpec((TILE_V, HIDDEN), lambda i: (i, 0)),
            pl.BlockSpec((TILE_V,), lambda i: (i,)),
        ],
        out_specs=pl.BlockSpec(memory_space=pl.ANY),
        out_shape=jax.ShapeDtypeStruct((vocab, BATCH), jnp.float32),
        scratch_shapes=[
            pltpu.VMEM((NBUF, TILE_V, BATCH), jnp.float32),
            pltpu.SemaphoreType.DMA((NBUF,)),
        ],
    )(x, W, b)
    return out_t.T


# speedup vs baseline: 1.0125x; 1.0125x over previous
import functools

import jax
import jax.numpy as jnp
from jax import lax
from jax.experimental import pallas as pl
from jax.experimental.pallas import tpu as pltpu
from jax.experimental.pallas import tpu_sc as plsc

BATCH = 1024
HIDDEN = 128
TILE_V = 4096
NBUF = 3


def _sc_gather(input_ids, emb_table):
    info = plsc.get_sparse_core_info()
    nc, ns = info.num_cores, info.num_subcores
    nw = nc * ns
    b_per_w = BATCH // nw
    mesh = plsc.VectorSubcoreMesh(core_axis_name="c", subcore_axis_name="s", num_cores=1)

    @functools.partial(
        pl.kernel,
        mesh=mesh,
        out_type=jax.ShapeDtypeStruct((BATCH, HIDDEN), jnp.float32),
        scratch_types=[
            pltpu.VMEM((b_per_w,), jnp.int32),
            pltpu.VMEM((b_per_w, HIDDEN), jnp.float32),
            pltpu.SemaphoreType.DMA,
            pltpu.SemaphoreType.DMA,
            pltpu.SemaphoreType.DMA,
            pltpu.SemaphoreType.DMA,
        ],
    )
    def gather_k(idx_hbm, table_hbm, out_hbm, idx_v, rows_v,
                 sem0, sem1, sem2, sem3):
        wid = lax.axis_index("s") * nc + lax.axis_index("c")
        base = wid * b_per_w
        h = b_per_w // 2
        pltpu.sync_copy(idx_hbm.at[pl.ds(base, b_per_w)], idx_v)
        c0 = pltpu.async_copy(
            table_hbm.at[idx_v.at[pl.ds(0, h)]], rows_v.at[pl.ds(0, h)], sem0)
        c1 = pltpu.async_copy(
            table_hbm.at[idx_v.at[pl.ds(h, h)]], rows_v.at[pl.ds(h, h)], sem1)
        c0.wait()
        w0 = pltpu.async_copy(
            rows_v.at[pl.ds(0, h)], out_hbm.at[pl.ds(base, h)], sem2)
        c1.wait()
        w1 = pltpu.async_copy(
            rows_v.at[pl.ds(h, h)], out_hbm.at[pl.ds(base + h, h)], sem3)
        w0.wait()
        w1.wait()

    return gather_k(input_ids, emb_table)


def _rows_of(grid, tail, j):
    return tail if (tail and j == grid - 1) else TILE_V


def _mm_body(grid, tail, x_ref, w_ref, b_ref, out_hbm, obuf, sems):
    i = pl.program_id(0)
    slot = lax.rem(i, NBUF)

    @pl.when(i >= NBUF)
    def _():
        pltpu.make_async_copy(
            obuf.at[slot], out_hbm.at[pl.ds(0, TILE_V)], sems.at[slot]
        ).wait()

    obuf.at[slot][...] = lax.dot_general(
        w_ref[...], x_ref[...],
        dimension_numbers=(((1,), (1,)), ((), ())),
        preferred_element_type=jnp.float32,
    ) + b_ref[...][:, None]

    @pl.when(i < grid - 1)
    def _():
        pltpu.make_async_copy(
            obuf.at[slot],
            out_hbm.at[pl.ds(i * TILE_V, TILE_V)],
            sems.at[slot],
        ).start()

    @pl.when(i == grid - 1)
    def _():
        rows = _rows_of(grid, tail, grid - 1)
        pltpu.make_async_copy(
            obuf.at[slot].at[pl.ds(0, rows)],
            out_hbm.at[pl.ds(i * TILE_V, rows)],
            sems.at[slot],
        ).start()
        for j in range(max(grid - NBUF, 0), grid):
            r = _rows_of(grid, tail, j)
            pltpu.make_async_copy(
                obuf.at[j % NBUF].at[pl.ds(0, r)],
                out_hbm.at[pl.ds(0, r)],
                sems.at[j % NBUF],
            ).wait()


def kernel(input_ids, emb_table, W, b):
    ids = input_ids.astype(jnp.int32)
    x = _sc_gather(ids, emb_table)

    vocab = W.shape[0]
    grid = (vocab + TILE_V - 1) // TILE_V
    tail = vocab % TILE_V

    out_t = pl.pallas_call(
        functools.partial(_mm_body, grid, tail),
        grid=(grid,),
        in_specs=[
            pl.BlockSpec((BATCH, HIDDEN), lambda i: (0, 0)),
            pl.BlockSpec((TILE_V, HIDDEN), lambda i: (i, 0)),
            pl.BlockSpec((TILE_V,), lambda i: (i,)),
        ],
        out_specs=pl.BlockSpec(memory_space=pl.ANY),
        out_shape=jax.ShapeDtypeStruct((vocab, BATCH), jnp.float32),
        scratch_shapes=[
            pltpu.VMEM((NBUF, TILE_V, BATCH), jnp.float32),
            pltpu.SemaphoreType.DMA((NBUF,)),
        ],
    )(x, W, b)
    return out_t.T
